# baseline (speedup 1.0000x reference)
import functools

import jax
import jax.numpy as jnp
from jax import lax
from jax.experimental import pallas as pl
from jax.experimental.pallas import tpu as pltpu
from jax.experimental.pallas import tpu_sc as plsc

NUM_SC_CORES = 2
NUM_SUBCORES = 16
NUM_WORKERS = NUM_SC_CORES * NUM_SUBCORES

BATCH = 16384
EMBED = 32
PACK = 128 // EMBED
ROWS_PER_WORKER = BATCH // NUM_WORKERS
IDX_CHUNK = 128
NUM_CHUNKS = ROWS_PER_WORKER // IDX_CHUNK


def _sc_gather_packed(u4_tbl, i4_tbl, user_ids, item_ids):
    mesh = plsc.VectorSubcoreMesh(core_axis_name="c", subcore_axis_name="s")

    @functools.partial(
        pl.kernel,
        mesh=mesh,
        compiler_params=pltpu.CompilerParams(use_tc_tiling_on_sc=True),
        out_type=(
            jax.ShapeDtypeStruct((BATCH, 128), jnp.float32),
            jax.ShapeDtypeStruct((BATCH, 128), jnp.float32),
        ),
        scratch_types=[
            pltpu.VMEM((ROWS_PER_WORKER,), jnp.int32),
            pltpu.VMEM((ROWS_PER_WORKER,), jnp.int32),
            pltpu.VMEM((ROWS_PER_WORKER, 128), jnp.float32),
            pltpu.SemaphoreType.DMA,
        ],
    )
    def k(ut_hbm, it_hbm, uid_hbm, iid_hbm, u_out, i_out,
          uidx_v, iidx_v, rows_v, sem):
        wid = lax.axis_index("s") * NUM_SC_CORES + lax.axis_index("c")
        base = wid * ROWS_PER_WORKER
        pltpu.sync_copy(uid_hbm.at[pl.ds(base, ROWS_PER_WORKER)], uidx_v)
        pltpu.sync_copy(iid_hbm.at[pl.ds(base, ROWS_PER_WORKER)], iidx_v)
        for g in range(ROWS_PER_WORKER // 16):
            s = pl.ds(g * 16, 16)
            uidx_v[s] = uidx_v[s] >> 2
            iidx_v[s] = iidx_v[s] >> 2
        for tbl, idx_v, out in ((ut_hbm, uidx_v, u_out),
                                (it_hbm, iidx_v, i_out)):
            copies = []
            for j in range(NUM_CHUNKS):
                c = pl.ds(j * IDX_CHUNK, IDX_CHUNK)
                copies.append(pltpu.async_copy(
                    tbl.at[idx_v.at[c]], rows_v.at[c], sem))
            for cp in copies:
                cp.wait()
            pltpu.sync_copy(rows_v, out.at[pl.ds(base, ROWS_PER_WORKER)])

    return k(u4_tbl, i4_tbl, user_ids, item_ids)


def _depack(x4, rem):
    out = jnp.where(rem == 0, x4, 0.0)
    for kk in range(1, PACK):
        rolled = jnp.concatenate([x4[:, 32 * kk:], x4[:, :32 * kk]], axis=1)
        out = out + jnp.where(rem == kk, rolled, 0.0)
    return out


def _tc_body(u4_ref, i4_ref, f_ref, uid_ref, iid_ref, wf_ref, bfp_ref,
             wp_ref, b_ref, out_ref):
    g = lax.dot_general(f_ref[...], wf_ref[...], (((1,), (1,)), ((), ())),
                        preferred_element_type=jnp.float32)
    u = _depack(u4_ref[...], uid_ref[...])
    i = _depack(i4_ref[...], iid_ref[...])
    gpad = jnp.concatenate(
        [g, jnp.zeros((g.shape[0], 128 - EMBED), jnp.float32)], axis=1)
    acc = u * (i + gpad + bfp_ref[...])
    out = jnp.sum(acc * wp_ref[...], axis=1) + b_ref[0, 0]
    out_ref[...] = out[None, None, :]


def _tc_combine(u4, i4, f, rem_u, rem_i, Wf, bf_pad, W_pad, b2):
    nblk = 8
    blk = BATCH // nblk
    return pl.pallas_call(
        _tc_body,
        grid=(nblk,),
        in_specs=[
            pl.BlockSpec((blk, 128), lambda n: (n, 0)),
            pl.BlockSpec((blk, 128), lambda n: (n, 0)),
            pl.BlockSpec((blk, 100), lambda n: (n, 0)),
            pl.BlockSpec((blk, 128), lambda n: (n, 0)),
            pl.BlockSpec((blk, 128), lambda n: (n, 0)),
            pl.BlockSpec((EMBED, 100), lambda n: (0, 0)),
            pl.BlockSpec((1, 128), lambda n: (0, 0)),
            pl.BlockSpec((1, 128), lambda n: (0, 0)),
            pl.BlockSpec((1, 1), lambda n: (0, 0)),
        ],
        out_specs=pl.BlockSpec((1, 1, blk), lambda n: (n, 0, 0)),
        out_shape=jax.ShapeDtypeStruct((nblk, 1, blk), jnp.float32),
    )(u4, i4, f, rem_u, rem_i, Wf, bf_pad, W_pad, b2)


def kernel(user_ids, item_ids, features, user_table, item_table, Wf, bf, W, b):
    uid = user_ids.astype(jnp.int32)
    iid = item_ids.astype(jnp.int32)
    u4_tbl = user_table.reshape(-1, 128)
    i4_tbl = item_table.reshape(-1, 128)
    u4, i4 = _sc_gather_packed(u4_tbl, i4_tbl, uid, iid)
    bf_pad = jnp.pad(bf, (0, 128 - EMBED)).reshape(1, 128)
    w_pad = jnp.pad(W[0], (0, 128 - EMBED)).reshape(1, 128)
    rem_u = jnp.broadcast_to((uid & 3).reshape(BATCH, 1), (BATCH, 128))
    rem_i = jnp.broadcast_to((iid & 3).reshape(BATCH, 1), (BATCH, 128))
    out = _tc_combine(u4, i4, features, rem_u, rem_i, Wf, bf_pad, w_pad,
                      b.reshape(1, 1))
    return out.reshape(BATCH)


# speedup vs baseline: 1.0599x; 1.0599x over previous
import functools

import jax
import jax.numpy as jnp
from jax import lax
from jax.experimental import pallas as pl
from jax.experimental.pallas import tpu as pltpu
from jax.experimental.pallas import tpu_sc as plsc

NUM_SC_CORES = 2
NUM_SUBCORES = 16
NUM_WORKERS = NUM_SC_CORES * NUM_SUBCORES

BATCH = 16384
EMBED = 32
ROWS_PER_WORKER = BATCH // NUM_WORKERS
IDX_CHUNK = 128
NUM_CHUNKS = ROWS_PER_WORKER // IDX_CHUNK


def _sc_gather_one(table, ids):
    mesh = plsc.VectorSubcoreMesh(core_axis_name="c", subcore_axis_name="s")

    @functools.partial(
        pl.kernel,
        mesh=mesh,
        compiler_params=pltpu.CompilerParams(use_tc_tiling_on_sc=False),
        out_type=jax.ShapeDtypeStruct((BATCH, EMBED), jnp.float32),
        scratch_types=[
            pltpu.VMEM((ROWS_PER_WORKER,), jnp.int32),
            pltpu.VMEM((ROWS_PER_WORKER, EMBED), jnp.float32),
            pltpu.SemaphoreType.DMA,
        ],
    )
    def k(t_hbm, id_hbm, out, idx_v, rows_v, sem):
        wid = lax.axis_index("s") * NUM_SC_CORES + lax.axis_index("c")
        base = wid * ROWS_PER_WORKER
        pltpu.sync_copy(id_hbm.at[pl.ds(base, ROWS_PER_WORKER)], idx_v)
        copies = []
        for j in range(NUM_CHUNKS):
            idx = pl.ds(j * IDX_CHUNK, IDX_CHUNK)
            copies.append(pltpu.async_copy(
                t_hbm.at[idx_v.at[idx]], rows_v.at[idx], sem))
        for c in copies:
            c.wait()
        pltpu.sync_copy(rows_v, out.at[pl.ds(base, ROWS_PER_WORKER)])

    return k(table, ids)


def _tc_body(u_ref, i_ref, f_ref, wf_ref, bf_ref, w_ref, b_ref, out_ref):
    g_t = lax.dot_general(wf_ref[...], f_ref[...], (((1,), (0,)), ((), ())),
                          preferred_element_type=jnp.float32)
    inter = u_ref[...] * (i_ref[...] + g_t + bf_ref[...])
    out = lax.dot_general(w_ref[...], inter, (((1,), (0,)), ((), ())),
                          preferred_element_type=jnp.float32)
    out_ref[...] = out + b_ref[...]


def _tc_combine(u_t, i_t, f_t, Wf, bf2, W, b2):
    return pl.pallas_call(
        _tc_body,
        out_shape=jax.ShapeDtypeStruct((1, BATCH), jnp.float32),
    )(u_t, i_t, f_t, Wf, bf2, W, b2)


def kernel(user_ids, item_ids, features, user_table, item_table, Wf, bf, W, b):
    u = _sc_gather_one(user_table, user_ids.astype(jnp.int32))
    i = _sc_gather_one(item_table, item_ids.astype(jnp.int32))
    out = _tc_combine(u.T, i.T, features.T, Wf, bf.reshape(EMBED, 1), W,
                      b.reshape(1, 1))
    return out.reshape(BATCH)


# speedup vs baseline: 1.0690x; 1.0086x over previous
import functools

import jax
import jax.numpy as jnp
from jax import lax
from jax.experimental import pallas as pl
from jax.experimental.pallas import tpu as pltpu
from jax.experimental.pallas import tpu_sc as plsc

NUM_SC_CORES = 2
NUM_SUBCORES = 16
NUM_WORKERS = NUM_SC_CORES * NUM_SUBCORES

BATCH = 16384
EMBED = 32
ROWS_PER_WORKER = BATCH // NUM_WORKERS
IDX_CHUNK = 128
NUM_CHUNKS = ROWS_PER_WORKER // IDX_CHUNK


def _sc_gather_one(table_pad, ids):
    mesh = plsc.VectorSubcoreMesh(core_axis_name="c", subcore_axis_name="s")

    @functools.partial(
        pl.kernel,
        mesh=mesh,
        compiler_params=pltpu.CompilerParams(use_tc_tiling_on_sc=False),
        out_type=jax.ShapeDtypeStruct((BATCH, 128), jnp.float32),
        scratch_types=[
            pltpu.VMEM((ROWS_PER_WORKER,), jnp.int32),
            pltpu.VMEM((ROWS_PER_WORKER, 128), jnp.float32),
            pltpu.SemaphoreType.DMA,
        ],
    )
    def k(t_hbm, id_hbm, out, idx_v, rows_v, sem):
        wid = lax.axis_index("s") * NUM_SC_CORES + lax.axis_index("c")
        base = wid * ROWS_PER_WORKER
        pltpu.sync_copy(id_hbm.at[pl.ds(base, ROWS_PER_WORKER)], idx_v)
        copies = []
        for j in range(NUM_CHUNKS):
            idx = pl.ds(j * IDX_CHUNK, IDX_CHUNK)
            copies.append(pltpu.async_copy(
                t_hbm.at[idx_v.at[idx]], rows_v.at[idx], sem))
        for c in copies:
            c.wait()
        pltpu.sync_copy(rows_v, out.at[pl.ds(base, ROWS_PER_WORKER)])

    return k(table_pad, ids)


def _tc_body(u_ref, i_ref, f_ref, wf_ref, bf_ref, w_ref, b_ref, out_ref):
    g = lax.dot_general(f_ref[...], wf_ref[...], (((1,), (1,)), ((), ())),
                        preferred_element_type=jnp.float32)
    u = u_ref[...][:, :EMBED]
    i = i_ref[...][:, :EMBED]
    acc = u * (i + g + bf_ref[...])
    out = jnp.sum(acc * w_ref[...], axis=1) + b_ref[0, 0]
    out_ref[...] = out[None, None, :]


def _tc_combine(u4, i4, f, Wf, bf2, W, b2):
    nblk = 8
    blk = BATCH // nblk
    return pl.pallas_call(
        _tc_body,
        grid=(nblk,),
        in_specs=[
            pl.BlockSpec((blk, 128), lambda n: (n, 0)),
            pl.BlockSpec((blk, 128), lambda n: (n, 0)),
            pl.BlockSpec((blk, 100), lambda n: (n, 0)),
            pl.BlockSpec((EMBED, 100), lambda n: (0, 0)),
            pl.BlockSpec((1, EMBED), lambda n: (0, 0)),
            pl.BlockSpec((1, EMBED), lambda n: (0, 0)),
            pl.BlockSpec((1, 1), lambda n: (0, 0)),
        ],
        out_specs=pl.BlockSpec((1, 1, blk), lambda n: (n, 0, 0)),
        out_shape=jax.ShapeDtypeStruct((nblk, 1, blk), jnp.float32),
    )(u4, i4, f, Wf, bf2, W, b2)


def kernel(user_ids, item_ids, features, user_table, item_table, Wf, bf, W, b):
    ut_pad = jnp.pad(user_table, ((0, 0), (0, 128 - EMBED)))
    it_pad = jnp.pad(item_table, ((0, 0), (0, 128 - EMBED)))
    u = _sc_gather_one(ut_pad, user_ids.astype(jnp.int32))
    i = _sc_gather_one(it_pad, item_ids.astype(jnp.int32))
    out = _tc_combine(u, i, features, Wf, bf.reshape(1, EMBED), W,
                      b.reshape(1, 1))
    return out.reshape(BATCH)
